# Initial kernel scaffold; baseline (speedup 1.0000x reference)
import jax
import jax.numpy as jnp
from jax.experimental import pallas as pl


def kernel(x, batch, W, b):
    raise NotImplementedError("write your pallas kernel here")


# speedup vs baseline: 7.1564x; 7.1564x over previous
import jax
import jax.numpy as jnp
from jax.experimental import pallas as pl
from jax.experimental.pallas import tpu as pltpu

NUM_SEG = 1024
BLK = 1000
WIN = 256


def _body(starts_ref, fb_ref, x_ref, batch_ref, w_ref, b_ref, out_ref):
    bidx = pl.program_id(0)

    @pl.when(bidx == 0)
    def _():
        out_ref[...] = jnp.zeros_like(out_ref)

    xb = x_ref[...]
    xh = xb.astype(jnp.bfloat16)
    s = jax.lax.dot_general(xh, w_ref[...], (((1,), (0,)), ((), ())),
                            preferred_element_type=jnp.float32)
    wgt = jax.nn.sigmoid(s + b_ref[0, 0])
    xwh = (xb * wgt).astype(jnp.bfloat16)

    brow = batch_ref[0]
    start = pl.multiple_of(starts_ref[bidx], 8)
    fb = fb_ref[bidx]

    @pl.when(fb == 0)
    def _():
        col = jax.lax.broadcasted_iota(jnp.int32, (WIN, BLK), 0) + start
        oh = (col == brow).astype(jnp.bfloat16)
        part = jax.lax.dot_general(oh, xwh, (((1,), (0,)), ((), ())),
                                   preferred_element_type=jnp.float32)
        out_ref[pl.ds(start, WIN), :] += part

    @pl.when(fb != 0)
    def _():
        col = jax.lax.broadcasted_iota(jnp.int32, (NUM_SEG, BLK), 0)
        oh = (col == brow).astype(jnp.bfloat16)
        full = jax.lax.dot_general(oh, xwh, (((1,), (0,)), ((), ())),
                                   preferred_element_type=jnp.float32)
        out_ref[...] += full


def kernel(x, batch, W, b):
    n, f = x.shape
    nb = n // BLK
    firsts = batch[::BLK]
    lasts = batch[BLK - 1::BLK]
    starts = (jnp.minimum(firsts, NUM_SEG - WIN) // 8 * 8).astype(jnp.int32)
    fb = (lasts >= starts + WIN).astype(jnp.int32)
    batch3 = batch.reshape(nb, 1, BLK)
    Wh = W.astype(jnp.bfloat16)
    b2 = b.reshape(1, 1)

    grid_spec = pltpu.PrefetchScalarGridSpec(
        num_scalar_prefetch=2,
        grid=(nb,),
        in_specs=[
            pl.BlockSpec((BLK, f), lambda i, *_: (i, 0)),
            pl.BlockSpec((1, 1, BLK), lambda i, *_: (i, 0, 0)),
            pl.BlockSpec((f, 1), lambda i, *_: (0, 0)),
            pl.BlockSpec((1, 1), lambda i, *_: (0, 0)),
        ],
        out_specs=pl.BlockSpec((NUM_SEG, f), lambda i, *_: (0, 0)),
    )
    return pl.pallas_call(
        _body,
        grid_spec=grid_spec,
        out_shape=jax.ShapeDtypeStruct((NUM_SEG, f), jnp.float32),
        compiler_params=pltpu.CompilerParams(
            dimension_semantics=("arbitrary",)),
    )(starts, fb, x, batch3, Wh, b2)
